# baseline (speedup 1.0000x reference)
import functools

import jax
import jax.numpy as jnp
from jax import lax
from jax.experimental import pallas as pl
from jax.experimental.pallas import tpu as pltpu
from jax.experimental.pallas import tpu_sc as plsc

_TIME_DIM = 1000
_MAX_LEN = 200
_EMBED_DIM = 128
_BATCH = 1024
_D = _MAX_LEN * _EMBED_DIM

_NUM_WORKERS = 32
_ROWS_PER_WORKER = _BATCH // _NUM_WORKERS
_CHUNK = 2
_NCHUNK = _ROWS_PER_WORKER // _CHUNK

_mesh = plsc.VectorSubcoreMesh(core_axis_name="c", subcore_axis_name="s")


@functools.partial(
    pl.kernel,
    out_type=jax.ShapeDtypeStruct((_BATCH, _D), jnp.float32),
    mesh=_mesh,
    scratch_types=[
        pltpu.VMEM((_NCHUNK, _CHUNK), jnp.int32),
        pltpu.VMEM((_CHUNK, _D), jnp.float32),
        pltpu.VMEM((_CHUNK, _D), jnp.float32),
        pltpu.SemaphoreType.DMA,
        pltpu.SemaphoreType.DMA,
        pltpu.SemaphoreType.DMA,
        pltpu.SemaphoreType.DMA,
    ],
)
def _sc_gather(t_hbm, table_hbm, out_hbm, idx_v, buf0, buf1, sg0, sg1, so0, so1):
    wid = lax.axis_index("s") * 2 + lax.axis_index("c")
    base = wid * _ROWS_PER_WORKER
    pltpu.sync_copy(t_hbm.at[wid], idx_v)

    bufs = (buf0, buf1)
    gsems = (sg0, sg1)
    osems = (so0, so1)

    g_handles = [None] * _NCHUNK
    o_handles = [None] * _NCHUNK

    g_handles[0] = pltpu.async_copy(table_hbm.at[idx_v.at[0]], bufs[0], gsems[0])
    for c in range(_NCHUNK):
        cur = c & 1
        g_handles[c].wait()
        o_handles[c] = pltpu.async_copy(
            bufs[cur], out_hbm.at[pl.ds(base + c * _CHUNK, _CHUNK)], osems[cur]
        )
        if c + 1 < _NCHUNK:
            if c >= 1:
                o_handles[c - 1].wait()
            g_handles[c + 1] = pltpu.async_copy(
                table_hbm.at[idx_v.at[c + 1]], bufs[1 - cur], gsems[1 - cur]
            )
    o_handles[_NCHUNK - 2].wait()
    o_handles[_NCHUNK - 1].wait()


def kernel(t, pos_encoding):
    table = pos_encoding.reshape(_TIME_DIM, _D)
    t3 = t.astype(jnp.int32).reshape(_NUM_WORKERS, _NCHUNK, _CHUNK)
    out = _sc_gather(t3, table)
    return out.reshape(_BATCH, _MAX_LEN, _EMBED_DIM)


# speedup vs baseline: 2.6594x; 2.6594x over previous
import functools

import jax
import jax.numpy as jnp
from jax import lax
from jax.experimental import pallas as pl
from jax.experimental.pallas import tpu as pltpu
from jax.experimental.pallas import tpu_sc as plsc

_TIME_DIM = 1000
_MAX_LEN = 200
_EMBED_DIM = 128
_BATCH = 1024

_NUM_WORKERS = 32
_ROWS_PER_WORKER = _BATCH // _NUM_WORKERS
_CHUNK = 2
_NCHUNK = _ROWS_PER_WORKER // _CHUNK

_mesh = plsc.VectorSubcoreMesh(core_axis_name="c", subcore_axis_name="s")


@functools.partial(
    pl.kernel,
    out_type=jax.ShapeDtypeStruct((_BATCH, _MAX_LEN, _EMBED_DIM), jnp.float32),
    mesh=_mesh,
    scratch_types=[
        pltpu.VMEM((_NCHUNK, _CHUNK), jnp.int32),
        pltpu.VMEM((_CHUNK, _MAX_LEN, _EMBED_DIM), jnp.float32),
        pltpu.VMEM((_CHUNK, _MAX_LEN, _EMBED_DIM), jnp.float32),
        pltpu.SemaphoreType.DMA,
        pltpu.SemaphoreType.DMA,
        pltpu.SemaphoreType.DMA,
        pltpu.SemaphoreType.DMA,
    ],
)
def _sc_gather(t_hbm, table_hbm, out_hbm, idx_v, buf0, buf1, sg0, sg1, so0, so1):
    wid = lax.axis_index("s") * 2 + lax.axis_index("c")
    base = wid * _ROWS_PER_WORKER
    pltpu.sync_copy(t_hbm.at[wid], idx_v)

    bufs = (buf0, buf1)
    gsems = (sg0, sg1)
    osems = (so0, so1)

    g_handles = [None] * _NCHUNK
    o_handles = [None] * _NCHUNK

    g_handles[0] = pltpu.async_copy(table_hbm.at[idx_v.at[0]], bufs[0], gsems[0])
    for c in range(_NCHUNK):
        cur = c & 1
        g_handles[c].wait()
        o_handles[c] = pltpu.async_copy(
            bufs[cur], out_hbm.at[pl.ds(base + c * _CHUNK, _CHUNK)], osems[cur]
        )
        if c + 1 < _NCHUNK:
            if c >= 1:
                o_handles[c - 1].wait()
            g_handles[c + 1] = pltpu.async_copy(
                table_hbm.at[idx_v.at[c + 1]], bufs[1 - cur], gsems[1 - cur]
            )
    o_handles[_NCHUNK - 2].wait()
    o_handles[_NCHUNK - 1].wait()


def kernel(t, pos_encoding):
    t3 = t.astype(jnp.int32).reshape(_NUM_WORKERS, _NCHUNK, _CHUNK)
    return _sc_gather(t3, pos_encoding)
